# baseline (speedup 1.0000x reference)
import numpy as np

import jax
import jax.numpy as jnp
from jax import lax
from jax.experimental import pallas as pl

P = 128
PTN = 64
N = P * PTN
R2 = 0.3 ** 2
K = 9
NEG = -jnp.inf
B = 4
G1 = P // B


def _new_points_np():
    xg = np.linspace(-0.2, 0.2, 4)
    yg = np.linspace(-0.2, 0.2, 4)
    xy = np.meshgrid(xg, yg)
    pts = np.array(xy).reshape(2, -1).T
    return np.concatenate([pts, np.zeros((pts.shape[0], 1))], axis=1).astype(np.float32)


def _plane_matrix_np():
    npts = _new_points_np()
    M = np.zeros((9, 48), dtype=np.float32)
    for j in range(16):
        for c in range(3):
            for d in range(3):
                M[3 * c + d, j * 3 + c] = npts[j, d]
    return M


def _dot(a, b):
    return jnp.dot(a, b, preferred_element_type=jnp.float32)


def _bdot(a, b):
    return lax.dot_general(a, b, (((2,), (1,)), ((0,), (0,))),
                           preferred_element_type=jnp.float32)


def _main_kernel(pos_ref, lf_ref,
                 ws1_ref, bs1_ref, ws2_ref, bs2_ref, ws3_ref, bs3_ref,
                 wc1s_ref, wc1d_ref, bc1_ref, wc2_ref, bc2_ref,
                 wm1l_ref, wm1p_ref, bm1_ref, wm2_ref, bm2_ref, wm3_ref, bm3_ref,
                 wm3r_ref, bm3t_ref,
                 rot9_ref, rotc_ref, posc_ref):
    pos2 = pos_ref[0]
    lf = lf_ref[0]

    lf_b = lax.broadcast_in_dim(lf, (B, PTN, 768), (0, 2)).reshape(B * PTN, 768)
    h = jax.nn.relu(_dot(jnp.concatenate([lf_b, pos2], axis=1), ws1_ref[...])
                    + bs1_ref[...])
    h = jax.nn.relu(_dot(h, ws2_ref[...]) + bs2_ref[...])
    h = jax.nn.relu(_dot(h, ws3_ref[...]) + bs3_ref[...])
    npos2 = jnp.tanh(h) + pos2

    gp = jnp.concatenate([pos2.reshape(B, PTN, 3), npos2.reshape(B, PTN, 3)],
                         axis=1)

    gpT = jnp.swapaxes(gp, 1, 2)
    dx = gp[:, :, 0:1] - gpT[:, 0:1, :]
    dy = gp[:, :, 1:2] - gpT[:, 1:2, :]
    dz = gp[:, :, 2:3] - gpT[:, 2:3, :]
    d2 = (dx * dx + dy * dy) + dz * dz
    iota_l = lax.broadcasted_iota(jnp.int32, (B, 128, 128), 2)
    keyi = jnp.bitwise_or(
        lax.shift_left(lax.bitcast_convert_type(d2, jnp.int32), 1),
        lax.shift_right_logical(iota_l, 6)) + 0x02000000
    cur = jnp.where(d2 <= R2, lax.bitcast_convert_type(keyi, jnp.float32),
                    jnp.inf)

    gpA = _bdot(gp, lax.broadcast_in_dim(wc1s_ref[...], (B, 3, 64), (1, 2)))
    posiB = _bdot(gp, lax.broadcast_in_dim(wc1d_ref[...], (B, 3, 64), (1, 2)))
    posiBc = bc1_ref[...] - posiB
    wc2 = wc2_ref[...]
    bc2 = bc2_ref[...]

    aggr = None
    for k in range(K):
        m = jnp.min(cur, axis=2, keepdims=True)
        selb = cur == m
        sel = selb.astype(jnp.float32)
        valid = m < jnp.inf
        cur = jnp.where(selb, jnp.inf, cur)
        hmsg = jax.nn.relu(_bdot(sel, gpA) + posiBc)
        msg = jax.nn.relu(lax.dot_general(
            hmsg, wc2, (((2,), (0,)), ((), ())),
            preferred_element_type=jnp.float32) + bc2)
        msg = jnp.where(valid, msg, NEG)
        aggr = msg if aggr is None else jnp.maximum(aggr, msg)

    aggr2 = aggr.reshape(B * 128, 128)
    lf_m = _dot(lf, wm1l_ref[...])
    lf_m3 = lax.broadcast_in_dim(lf_m, (B, 128, 512), (0, 2)).reshape(B * 128, 512)
    h1 = jax.nn.relu(_dot(aggr2, wm1p_ref[...]) + lf_m3 + bm1_ref[...])
    h2 = jax.nn.relu(_dot(h1, wm2_ref[...]) + bm2_ref[...])
    r9 = jax.nn.relu(_dot(h2, wm3_ref[...]) + bm3_ref[...])

    r9t = jax.nn.relu(lax.dot_general(wm3r_ref[...], h2, (((1,), (1,)), ((), ())),
                                      preferred_element_type=jnp.float32)
                      + bm3t_ref[...])
    rows = [r9t[i:i + 1, :] for i in range(9)]
    rc_rows = []
    for a in range(3):
        for b in range(3):
            rc_rows.append(rows[a] * rows[b] + rows[3 + a] * rows[3 + b]
                           + rows[6 + a] * rows[6 + b])
    rotc_ref[0] = jnp.concatenate(rc_rows, axis=0)

    r93 = r9.reshape(B, 128, 9)
    rot9_ref[0] = r93[:, :PTN].reshape(B * PTN, 9)
    rot9_ref[1] = r93[:, PTN:].reshape(B * PTN, 9)
    posc_ref[0] = pos2
    posc_ref[1] = npos2


def _plane_kernel(rot9_ref, c48_ref, m_ref, out_ref):
    out_ref[0] = _dot(rot9_ref[...], m_ref[...]) + c48_ref[...]


def kernel(x, pos, batch, diff, local_fea, shift_mlp, main_mlp, conv_mlp):
    del x, batch, diff
    (ws1, bs1), (ws2, bs2), (ws3, bs3) = shift_mlp
    (wm1, bm1), (wm2, bm2), (wm3, bm3) = main_mlp
    (wc1, bc1), (wc2, bc2) = conv_mlp

    pos3 = pos.reshape(G1, B * PTN, 3)
    lf3 = local_fea.reshape(G1, B, 768)

    args = [
        pos3, lf3,
        ws1.T, bs1.reshape(1, -1),
        ws2.T, bs2.reshape(1, -1), ws3.T, bs3.reshape(1, -1),
        wc1[:, :3].T + wc1[:, 3:].T, wc1[:, 3:].T, bc1.reshape(1, 1, -1),
        wc2.T, bc2.reshape(1, -1),
        wm1[:, :768].T, wm1[:, 768:].T, bm1.reshape(1, -1),
        wm2.T, bm2.reshape(1, -1), wm3.T, bm3.reshape(1, -1),
        wm3, bm3.reshape(-1, 1),
    ]

    def fullspec(a):
        nd = a.ndim
        return pl.BlockSpec(a.shape, lambda p, _n=nd: (0,) * _n)

    in_specs = [
        pl.BlockSpec((1, B * PTN, 3), lambda p: (p, 0, 0)),
        pl.BlockSpec((1, B, 768), lambda p: (p, 0, 0)),
    ] + [fullspec(a) for a in args[2:]]

    rot9, rotc, posc = pl.pallas_call(
        _main_kernel,
        grid=(G1,),
        in_specs=in_specs,
        out_specs=[
            pl.BlockSpec((2, B * PTN, 9), lambda p: (0, p, 0)),
            pl.BlockSpec((1, 9, B * 128), lambda p: (p, 0, 0)),
            pl.BlockSpec((2, B * PTN, 3), lambda p: (0, p, 0)),
        ],
        out_shape=[
            jax.ShapeDtypeStruct((2, N, 9), jnp.float32),
            jax.ShapeDtypeStruct((G1, 9, B * 128), jnp.float32),
            jax.ShapeDtypeStruct((2, N, 3), jnp.float32),
        ],
    )(*args)

    rot9f = rot9.reshape(2 * N, 9)
    c48 = posc.reshape(1024, 48)
    M = jnp.asarray(_plane_matrix_np())

    plane = pl.pallas_call(
        _plane_kernel,
        grid=(16,),
        in_specs=[
            pl.BlockSpec((1024, 9), lambda p: (p, 0)),
            pl.BlockSpec((1024, 48), lambda p: (0, 0)),
            pl.BlockSpec((9, 48), lambda p: (0, 0)),
        ],
        out_specs=pl.BlockSpec((1, 1024, 48), lambda p: (p, 0, 0)),
        out_shape=jax.ShapeDtypeStruct((16, 1024, 48), jnp.float32),
    )(rot9f, c48, M)

    plane_init = plane.reshape(P, 2048, 3)
    rt = rotc.transpose(0, 2, 1).reshape(P, 128, 9)
    rot_constrain = jnp.concatenate(
        [rt[:, :PTN].reshape(N, 9), rt[:, PTN:].reshape(N, 9)],
        axis=0).reshape(2 * N, 3, 3)
    return plane_init, rot_constrain


# speedup vs baseline: 1.0480x; 1.0480x over previous
import numpy as np

import jax
import jax.numpy as jnp
from jax import lax
from jax.experimental import pallas as pl

P = 128
PTN = 64
N = P * PTN
R2 = 0.3 ** 2
K = 9
NEG = -jnp.inf
B = 8
G1 = P // B


def _new_points_np():
    xg = np.linspace(-0.2, 0.2, 4)
    yg = np.linspace(-0.2, 0.2, 4)
    xy = np.meshgrid(xg, yg)
    pts = np.array(xy).reshape(2, -1).T
    return np.concatenate([pts, np.zeros((pts.shape[0], 1))], axis=1).astype(np.float32)


def _plane_matrix_np():
    npts = _new_points_np()
    M = np.zeros((9, 48), dtype=np.float32)
    for j in range(16):
        for c in range(3):
            for d in range(3):
                M[3 * c + d, j * 3 + c] = npts[j, d]
    return M


def _dot(a, b):
    return jnp.dot(a, b, preferred_element_type=jnp.float32)


def _bdot(a, b):
    return lax.dot_general(a, b, (((2,), (1,)), ((0,), (0,))),
                           preferred_element_type=jnp.float32)


def _main_kernel(pos_ref, lf_ref,
                 ws1_ref, bs1_ref, ws2_ref, bs2_ref, ws3_ref, bs3_ref,
                 wc1s_ref, wc1d_ref, bc1_ref, wc2_ref, bc2_ref,
                 wm1l_ref, wm1p_ref, bm1_ref, wm2_ref, bm2_ref, wm3_ref, bm3_ref,
                 wm3r_ref, bm3t_ref,
                 rot9_ref, rotc_ref, posc_ref):
    pos2 = pos_ref[0]
    lf = lf_ref[0]

    lf_b = lax.broadcast_in_dim(lf, (B, PTN, 768), (0, 2)).reshape(B * PTN, 768)
    h = jax.nn.relu(_dot(jnp.concatenate([lf_b, pos2], axis=1), ws1_ref[...])
                    + bs1_ref[...])
    h = jax.nn.relu(_dot(h, ws2_ref[...]) + bs2_ref[...])
    h = jax.nn.relu(_dot(h, ws3_ref[...]) + bs3_ref[...])
    npos2 = jnp.tanh(h) + pos2

    gp = jnp.concatenate([pos2.reshape(B, PTN, 3), npos2.reshape(B, PTN, 3)],
                         axis=1)

    gpT = jnp.swapaxes(gp, 1, 2)
    dx = gp[:, :, 0:1] - gpT[:, 0:1, :]
    dy = gp[:, :, 1:2] - gpT[:, 1:2, :]
    dz = gp[:, :, 2:3] - gpT[:, 2:3, :]
    d2 = (dx * dx + dy * dy) + dz * dz
    iota_l = lax.broadcasted_iota(jnp.int32, (B, 128, 128), 2)
    keyi = jnp.bitwise_or(
        lax.shift_left(lax.bitcast_convert_type(d2, jnp.int32), 1),
        lax.shift_right_logical(iota_l, 6)) + 0x02000000
    cur = jnp.where(d2 <= R2, lax.bitcast_convert_type(keyi, jnp.float32),
                    jnp.inf)

    gpA = _bdot(gp, lax.broadcast_in_dim(wc1s_ref[...], (B, 3, 64), (1, 2)))
    posiB = _bdot(gp, lax.broadcast_in_dim(wc1d_ref[...], (B, 3, 64), (1, 2)))
    posiBc = bc1_ref[...] - posiB
    wc2 = wc2_ref[...]
    bc2 = bc2_ref[...]

    aggr = None
    for k in range(K):
        m = jnp.min(cur, axis=2, keepdims=True)
        selb = cur == m
        sel = selb.astype(jnp.float32)
        valid = m < jnp.inf
        cur = jnp.where(selb, jnp.inf, cur)
        hmsg = jax.nn.relu(_bdot(sel, gpA) + posiBc)
        msg = jax.nn.relu(lax.dot_general(
            hmsg, wc2, (((2,), (0,)), ((), ())),
            preferred_element_type=jnp.float32) + bc2)
        msg = jnp.where(valid, msg, NEG)
        aggr = msg if aggr is None else jnp.maximum(aggr, msg)

    aggr2 = aggr.reshape(B * 128, 128)
    lf_m = _dot(lf, wm1l_ref[...])
    lf_m3 = lax.broadcast_in_dim(lf_m, (B, 128, 512), (0, 2)).reshape(B * 128, 512)
    h1 = jax.nn.relu(_dot(aggr2, wm1p_ref[...]) + lf_m3 + bm1_ref[...])
    h2 = jax.nn.relu(_dot(h1, wm2_ref[...]) + bm2_ref[...])
    r9 = jax.nn.relu(_dot(h2, wm3_ref[...]) + bm3_ref[...])

    r9t = jax.nn.relu(lax.dot_general(wm3r_ref[...], h2, (((1,), (1,)), ((), ())),
                                      preferred_element_type=jnp.float32)
                      + bm3t_ref[...])
    rows = [r9t[i:i + 1, :] for i in range(9)]
    rc_rows = []
    for a in range(3):
        for b in range(3):
            rc_rows.append(rows[a] * rows[b] + rows[3 + a] * rows[3 + b]
                           + rows[6 + a] * rows[6 + b])
    rotc_ref[0] = jnp.concatenate(rc_rows, axis=0)

    r93 = r9.reshape(B, 128, 9)
    rot9_ref[0] = r93[:, :PTN].reshape(B * PTN, 9)
    rot9_ref[1] = r93[:, PTN:].reshape(B * PTN, 9)
    posc_ref[0] = pos2
    posc_ref[1] = npos2


def _plane_kernel(rot9_ref, c48_ref, m_ref, out_ref):
    out_ref[0] = _dot(rot9_ref[...], m_ref[...]) + c48_ref[...]


def kernel(x, pos, batch, diff, local_fea, shift_mlp, main_mlp, conv_mlp):
    del x, batch, diff
    (ws1, bs1), (ws2, bs2), (ws3, bs3) = shift_mlp
    (wm1, bm1), (wm2, bm2), (wm3, bm3) = main_mlp
    (wc1, bc1), (wc2, bc2) = conv_mlp

    pos3 = pos.reshape(G1, B * PTN, 3)
    lf3 = local_fea.reshape(G1, B, 768)

    args = [
        pos3, lf3,
        ws1.T, bs1.reshape(1, -1),
        ws2.T, bs2.reshape(1, -1), ws3.T, bs3.reshape(1, -1),
        wc1[:, :3].T + wc1[:, 3:].T, wc1[:, 3:].T, bc1.reshape(1, 1, -1),
        wc2.T, bc2.reshape(1, -1),
        wm1[:, :768].T, wm1[:, 768:].T, bm1.reshape(1, -1),
        wm2.T, bm2.reshape(1, -1), wm3.T, bm3.reshape(1, -1),
        wm3, bm3.reshape(-1, 1),
    ]

    def fullspec(a):
        nd = a.ndim
        return pl.BlockSpec(a.shape, lambda p, _n=nd: (0,) * _n)

    in_specs = [
        pl.BlockSpec((1, B * PTN, 3), lambda p: (p, 0, 0)),
        pl.BlockSpec((1, B, 768), lambda p: (p, 0, 0)),
    ] + [fullspec(a) for a in args[2:]]

    rot9, rotc, posc = pl.pallas_call(
        _main_kernel,
        grid=(G1,),
        in_specs=in_specs,
        out_specs=[
            pl.BlockSpec((2, B * PTN, 9), lambda p: (0, p, 0)),
            pl.BlockSpec((1, 9, B * 128), lambda p: (p, 0, 0)),
            pl.BlockSpec((2, B * PTN, 3), lambda p: (0, p, 0)),
        ],
        out_shape=[
            jax.ShapeDtypeStruct((2, N, 9), jnp.float32),
            jax.ShapeDtypeStruct((G1, 9, B * 128), jnp.float32),
            jax.ShapeDtypeStruct((2, N, 3), jnp.float32),
        ],
    )(*args)

    rot9f = rot9.reshape(2 * N, 9)
    c48 = posc.reshape(1024, 48)
    M = jnp.asarray(_plane_matrix_np())

    plane = pl.pallas_call(
        _plane_kernel,
        grid=(16,),
        in_specs=[
            pl.BlockSpec((1024, 9), lambda p: (p, 0)),
            pl.BlockSpec((1024, 48), lambda p: (0, 0)),
            pl.BlockSpec((9, 48), lambda p: (0, 0)),
        ],
        out_specs=pl.BlockSpec((1, 1024, 48), lambda p: (p, 0, 0)),
        out_shape=jax.ShapeDtypeStruct((16, 1024, 48), jnp.float32),
    )(rot9f, c48, M)

    plane_init = plane.reshape(P, 2048, 3)
    rt = rotc.transpose(0, 2, 1).reshape(P, 128, 9)
    rot_constrain = jnp.concatenate(
        [rt[:, :PTN].reshape(N, 9), rt[:, PTN:].reshape(N, 9)],
        axis=0).reshape(2 * N, 3, 3)
    return plane_init, rot_constrain
